# baseline (speedup 1.0000x reference)
import functools

import jax
import jax.numpy as jnp
from jax import lax
from jax.experimental import pallas as pl
from jax.experimental.pallas import tpu as pltpu
from jax.experimental.pallas import tpu_sc as plsc

MEM_TOKENS = 50
DIM = 64

NC = 2
NS = 16
NW = NC * NS

CHUNK = 128


def _sc_gather(idx_hbm, table_hbm, out_hbm, idx_v, rows_v, sem):
    n = idx_hbm.shape[0]
    per_w = n // NW
    nchunk = per_w // CHUNK
    wid = lax.axis_index("s") * NC + lax.axis_index("c")
    base = wid * per_w
    pltpu.sync_copy(idx_hbm.at[pl.ds(base, per_w)], idx_v)

    def body(c, _):
        pltpu.async_copy(
            table_hbm.at[idx_v.at[pl.ds(c * CHUNK, CHUNK)]], rows_v, sem
        ).wait()
        pltpu.sync_copy(rows_v, out_hbm.at[pl.ds(base + c * CHUNK, CHUNK)])
        return 0

    lax.fori_loop(0, nchunk, body, 0)


def _tc_matmul(h_ref, idx_ref, wt_ref, o_ref):
    m = (idx_ref[...] != 0).astype(jnp.float32)
    h = h_ref[...] * m
    o_ref[...] = jnp.dot(h, wt_ref[...], preferred_element_type=jnp.float32)


def kernel(x, emb_table, W_proj):
    B, L = x.shape
    n = B * MEM_TOKENS
    idx = x[:, -MEM_TOKENS:].reshape(n).astype(jnp.int32)

    mesh = plsc.VectorSubcoreMesh(core_axis_name="c", subcore_axis_name="s")
    per_w = n // NW
    gather = pl.kernel(
        _sc_gather,
        mesh=mesh,
        out_type=jax.ShapeDtypeStruct((n, DIM), jnp.float32),
        scratch_types=[
            pltpu.VMEM((per_w,), jnp.int32),
            pltpu.VMEM((CHUNK, DIM), jnp.float32),
            pltpu.SemaphoreType.DMA,
        ],
        compiler_params=pltpu.CompilerParams(use_tc_tiling_on_sc=False),
    )
    rows = gather(idx, emb_table)

    BM = 2048
    out = pl.pallas_call(
        _tc_matmul,
        grid=(n // BM,),
        in_specs=[
            pl.BlockSpec((BM, DIM), lambda i: (i, 0)),
            pl.BlockSpec((BM, 1), lambda i: (i, 0)),
            pl.BlockSpec((DIM, DIM), lambda i: (0, 0)),
        ],
        out_specs=pl.BlockSpec((BM, DIM), lambda i: (i, 0)),
        out_shape=jax.ShapeDtypeStruct((n, DIM), jnp.float32),
    )(rows, idx.reshape(n, 1), W_proj.T)

    return out.reshape(B, MEM_TOKENS, DIM)


# speedup vs baseline: 1.0108x; 1.0108x over previous
import functools

import jax
import jax.numpy as jnp
from jax import lax
from jax.experimental import pallas as pl
from jax.experimental.pallas import tpu as pltpu
from jax.experimental.pallas import tpu_sc as plsc

MEM_TOKENS = 50
DIM = 64

NC = 2
NS = 16
NW = NC * NS

CHUNK = 128
RING = 10
PREF = 5


def _sc_gather(idx_hbm, table_hbm, out_hbm, idx_v, ring_v, gsem, wsem):
    n = idx_hbm.shape[0]
    per_w = n // NW
    nchunk = per_w // CHUNK
    wid = lax.axis_index("s") * NC + lax.axis_index("c")
    base = wid * per_w
    pltpu.sync_copy(idx_hbm.at[pl.ds(base, per_w)], idx_v)

    def gather(c, b):
        return pltpu.make_async_copy(
            table_hbm.at[idx_v.at[pl.ds(c * CHUNK, CHUNK)]],
            ring_v.at[b], gsem.at[b])

    def write(c, b):
        return pltpu.make_async_copy(
            ring_v.at[b], out_hbm.at[pl.ds(base + c * CHUNK, CHUNK)],
            wsem.at[b])

    for b in range(PREF):
        gather(b, b).start()

    def body(i, _):
        for b in range(RING):
            s = i * RING + b
            bg = (b + PREF) % RING

            @pl.when(s + PREF < nchunk)
            def _():
                @pl.when(s >= PREF)
                def _():
                    write(s - PREF, bg).wait()
                gather(s + PREF, bg).start()

            gather(s, b).wait()
            write(s, b).start()
        return 0

    lax.fori_loop(0, nchunk // RING, body, 0, unroll=False)

    for b in range(RING):
        c = nchunk - RING + b
        write(c, c % RING).wait()


def _tc_matmul(h_ref, idx_ref, wt_ref, o_ref):
    m = (idx_ref[...] != 0).astype(jnp.float32)
    h = h_ref[...] * m
    o_ref[...] = jnp.dot(h, wt_ref[...], preferred_element_type=jnp.float32)


def kernel(x, emb_table, W_proj):
    B, L = x.shape
    n = B * MEM_TOKENS
    idx = x[:, -MEM_TOKENS:].reshape(n).astype(jnp.int32)

    mesh = plsc.VectorSubcoreMesh(core_axis_name="c", subcore_axis_name="s")
    per_w = n // NW
    gather = pl.kernel(
        _sc_gather,
        mesh=mesh,
        out_type=jax.ShapeDtypeStruct((n, DIM), jnp.float32),
        scratch_types=[
            pltpu.VMEM((per_w,), jnp.int32),
            pltpu.VMEM((RING, CHUNK, DIM), jnp.float32),
            pltpu.SemaphoreType.DMA((RING,)),
            pltpu.SemaphoreType.DMA((RING,)),
        ],
        compiler_params=pltpu.CompilerParams(use_tc_tiling_on_sc=False),
    )
    rows = gather(idx, emb_table)

    BM = 2048
    out = pl.pallas_call(
        _tc_matmul,
        grid=(n // BM,),
        in_specs=[
            pl.BlockSpec((BM, DIM), lambda i: (i, 0)),
            pl.BlockSpec((BM, 1), lambda i: (i, 0)),
            pl.BlockSpec((DIM, DIM), lambda i: (0, 0)),
        ],
        out_specs=pl.BlockSpec((BM, DIM), lambda i: (i, 0)),
        out_shape=jax.ShapeDtypeStruct((n, DIM), jnp.float32),
    )(rows, idx.reshape(n, 1), W_proj.T)

    return out.reshape(B, MEM_TOKENS, DIM)


# speedup vs baseline: 1.9730x; 1.9518x over previous
import jax
import jax.numpy as jnp
from jax import lax
from jax.experimental import pallas as pl
from jax.experimental.pallas import tpu as pltpu
from jax.experimental.pallas import tpu_sc as plsc

MEM_TOKENS = 50
DIM = 64
GRP = 8

NC = 2
NS = 16
NW = NC * NS

CHUNK = 64


def _sc_gather(idx_hbm, table_hbm, out_hbm, idx_v, obuf, dsem, wsem):
    n = idx_hbm.shape[0]
    per_w = n // NW
    nchunk = per_w // CHUNK
    wid = lax.axis_index("s") * NC + lax.axis_index("c")
    base = wid * per_w
    pltpu.sync_copy(idx_hbm.at[pl.ds(base, per_w)], idx_v)

    iota16 = lax.iota(jnp.int32, 16)
    zeros16 = jnp.zeros((16,), jnp.float32)

    def fire(c, b):
        for g in range(CHUNK // 16):
            idx16 = idx_v[pl.ds(c * CHUNK + g * 16, 16)]
            for l in range(16):
                s = jnp.sum(jnp.where(iota16 == l, idx16, 0))
                gi = lax.shift_right_logical(s, 3)
                ri = lax.bitwise_and(s, 7)
                pltpu.make_async_copy(
                    table_hbm.at[gi, pl.ds(ri, 1)],
                    obuf.at[b, pl.ds(g * 16 + l, 1)],
                    dsem.at[b]).start()

    def drain(b):
        pltpu.make_async_copy(
            out_hbm.at[pl.ds(0, CHUNK)], obuf.at[b], dsem.at[b]).wait()

    def mask_pass(c, b):
        nz_total = jnp.sum(jnp.where(idx_v[pl.ds(c * CHUNK, 16)] == 0, 1, 0))
        for g in range(CHUNK // 16):
            if g > 0:
                nz_total = nz_total + jnp.sum(jnp.where(
                    idx_v[pl.ds(c * CHUNK + g * 16, 16)] == 0, 1, 0))

        @pl.when(nz_total > 0)
        def _():
            for g in range(CHUNK // 16):
                idx16 = idx_v[pl.ds(c * CHUNK + g * 16, 16)]
                z = idx16 == 0
                tok16 = iota16 + (g * 16)
                for c0 in range(DIM):
                    col = jnp.full((16,), c0, jnp.int32)
                    plsc.store_scatter(obuf.at[b], [tok16, col], zeros16,
                                       mask=z)

    def write(c, b):
        return pltpu.make_async_copy(
            obuf.at[b], out_hbm.at[pl.ds(base + c * CHUNK, CHUNK)],
            wsem.at[b])

    fire(0, 0)

    def body(i, _):
        for b in range(2):
            c = i * 2 + b
            nb = 1 - b

            @pl.when(c + 1 < nchunk)
            def _():
                @pl.when(c >= 1)
                def _():
                    write(c - 1, nb).wait()
                fire(c + 1, nb)

            drain(b)
            mask_pass(c, b)
            write(c, b).start()
        return 0

    lax.fori_loop(0, nchunk // 2, body, 0)

    write(nchunk - 2, 0).wait()
    write(nchunk - 1, 1).wait()


def _tc_matmul(h_ref, wt_ref, o_ref):
    nb = o_ref.shape[0]
    acc = jnp.dot(h_ref[...], wt_ref[...], preferred_element_type=jnp.float32)
    o_ref[...] = acc.reshape(nb, MEM_TOKENS, DIM)


def kernel(x, emb_table, W_proj):
    B, L = x.shape
    n = B * MEM_TOKENS
    idx = x[:, -MEM_TOKENS:].reshape(n).astype(jnp.int32)
    table3 = emb_table.reshape(emb_table.shape[0] // GRP, GRP, DIM)

    mesh = plsc.VectorSubcoreMesh(core_axis_name="c", subcore_axis_name="s")
    per_w = n // NW
    gather = pl.kernel(
        _sc_gather,
        mesh=mesh,
        out_type=jax.ShapeDtypeStruct((n, DIM), jnp.float32),
        scratch_types=[
            pltpu.VMEM((per_w,), jnp.int32),
            pltpu.VMEM((2, CHUNK, DIM), jnp.float32),
            pltpu.SemaphoreType.DMA((2,)),
            pltpu.SemaphoreType.DMA((2,)),
        ],
        compiler_params=pltpu.CompilerParams(needs_layout_passes=False),
    )
    rows = gather(idx, table3)

    BB = 32
    out = pl.pallas_call(
        _tc_matmul,
        grid=(B // BB,),
        in_specs=[
            pl.BlockSpec((BB * MEM_TOKENS, DIM), lambda i: (i, 0)),
            pl.BlockSpec((DIM, DIM), lambda i: (0, 0)),
        ],
        out_specs=pl.BlockSpec((BB, MEM_TOKENS, DIM), lambda i: (i, 0, 0)),
        out_shape=jax.ShapeDtypeStruct((B, MEM_TOKENS, DIM), jnp.float32),
    )(rows, W_proj.T)

    return out


# speedup vs baseline: 2.1180x; 1.0735x over previous
import jax
import jax.numpy as jnp
from jax import lax
from jax.experimental import pallas as pl
from jax.experimental.pallas import tpu as pltpu
from jax.experimental.pallas import tpu_sc as plsc

MEM_TOKENS = 50
DIM = 64
GRP = 8

NC = 2
NS = 16
NW = NC * NS

CHUNK = 64


def _sc_gather(idx_hbm, table_hbm, out_hbm, idx_v, obuf, dsem, wsem):
    n = idx_hbm.shape[0]
    per_w = n // NW
    nchunk = per_w // CHUNK
    wid = lax.axis_index("s") * NC + lax.axis_index("c")
    base = wid * per_w
    pltpu.sync_copy(idx_hbm.at[pl.ds(base, per_w)], idx_v)

    iota16 = lax.iota(jnp.int32, 16)
    zeros16 = jnp.zeros((16,), jnp.float32)

    def fire(c, b):
        for g in range(CHUNK // 16):
            idx16 = idx_v[pl.ds(c * CHUNK + g * 16, 16)]
            for l in range(16):
                s = idx16[l]
                gi = lax.shift_right_logical(s, 3)
                ri = lax.bitwise_and(s, 7)
                pltpu.make_async_copy(
                    table_hbm.at[gi, pl.ds(ri, 1)],
                    obuf.at[b, pl.ds(g * 16 + l, 1)],
                    dsem.at[b]).start()

    def drain(b):
        pltpu.make_async_copy(
            out_hbm.at[pl.ds(0, CHUNK)], obuf.at[b], dsem.at[b]).wait()

    def mask_pass(c, b):
        nz_total = jnp.sum(jnp.where(idx_v[pl.ds(c * CHUNK, 16)] == 0, 1, 0))
        for g in range(1, CHUNK // 16):
            nz_total = nz_total + jnp.sum(jnp.where(
                idx_v[pl.ds(c * CHUNK + g * 16, 16)] == 0, 1, 0))

        @pl.when(nz_total > 0)
        def _():
            for g in range(CHUNK // 16):
                idx16 = idx_v[pl.ds(c * CHUNK + g * 16, 16)]
                z = idx16 == 0
                tok16 = iota16 + (g * 16)
                for c0 in range(DIM):
                    col = jnp.full((16,), c0, jnp.int32)
                    plsc.store_scatter(obuf.at[b], [tok16, col], zeros16,
                                       mask=z)

    def write(c, b):
        return pltpu.make_async_copy(
            obuf.at[b], out_hbm.at[pl.ds(base + c * CHUNK, CHUNK)],
            wsem.at[b])

    fire(0, 0)

    def body(i, _):
        for b in range(2):
            c = i * 2 + b
            nb = 1 - b

            @pl.when(c + 1 < nchunk)
            def _():
                @pl.when(c >= 1)
                def _():
                    write(c - 1, nb).wait()
                fire(c + 1, nb)

            drain(b)
            mask_pass(c, b)
            write(c, b).start()
        return 0

    lax.fori_loop(0, nchunk // 2, body, 0)

    write(nchunk - 2, 0).wait()
    write(nchunk - 1, 1).wait()


def _tc_matmul(h_ref, wt_ref, o_ref):
    nb = o_ref.shape[0]
    acc = jnp.dot(h_ref[...], wt_ref[...], preferred_element_type=jnp.float32)
    o_ref[...] = acc.reshape(nb, MEM_TOKENS, DIM)


def kernel(x, emb_table, W_proj):
    B, L = x.shape
    n = B * MEM_TOKENS
    idx = x[:, -MEM_TOKENS:].reshape(n).astype(jnp.int32)
    table3 = emb_table.reshape(emb_table.shape[0] // GRP, GRP, DIM)

    mesh = plsc.VectorSubcoreMesh(core_axis_name="c", subcore_axis_name="s")
    per_w = n // NW
    gather = pl.kernel(
        _sc_gather,
        mesh=mesh,
        out_type=jax.ShapeDtypeStruct((n, DIM), jnp.float32),
        scratch_types=[
            pltpu.VMEM((per_w,), jnp.int32),
            pltpu.VMEM((2, CHUNK, DIM), jnp.float32),
            pltpu.SemaphoreType.DMA((2,)),
            pltpu.SemaphoreType.DMA((2,)),
        ],
        compiler_params=pltpu.CompilerParams(needs_layout_passes=False),
    )
    rows = gather(idx, table3)

    BB = 64
    out = pl.pallas_call(
        _tc_matmul,
        grid=(B // BB,),
        in_specs=[
            pl.BlockSpec((BB * MEM_TOKENS, DIM), lambda i: (i, 0)),
            pl.BlockSpec((DIM, DIM), lambda i: (0, 0)),
        ],
        out_specs=pl.BlockSpec((BB, MEM_TOKENS, DIM), lambda i: (i, 0, 0)),
        out_shape=jax.ShapeDtypeStruct((B, MEM_TOKENS, DIM), jnp.float32),
    )(rows, W_proj.T)

    return out
